# baseline (speedup 1.0000x reference)
import functools

import jax
import jax.numpy as jnp
from jax import lax
from jax.experimental import pallas as pl
from jax.experimental.pallas import tpu as pltpu
from jax.experimental.pallas import tpu_sc as plsc

N = 10000
E = 320000
H = 128
P = 64
S = 156

_NC = 2
_NS = 16
_NW = _NC * _NS
_CHUNK = 128

N_ACC = 10112
E_PAD = 323584
NPART = P + 1
S_PAD = 160
PART_ACC = 128
PE_PAD = 12288


def _seg_sum_gather(d, n_acc, e_pad):
    chunks_per_tile = e_pad // (_NW * _CHUNK)
    rows_per_tile = n_acc // _NS
    mesh = plsc.VectorSubcoreMesh(core_axis_name="c", subcore_axis_name="s")

    @functools.partial(
        pl.kernel,
        out_type=jax.ShapeDtypeStruct((_NC, n_acc, d), jnp.float32),
        mesh=mesh,
        scratch_types=[
            pltpu.VMEM((_CHUNK,), jnp.int32),
            pltpu.VMEM((_CHUNK,), jnp.int32),
            pltpu.VMEM((_CHUNK, d), jnp.float32),
            pltpu.VMEM_SHARED((n_acc, d), jnp.float32),
        ],
    )
    def seg_sum(vals_hbm, src_hbm, dst_hbm, zeros_hbm, out_hbm,
                src_v, dst_v, rows_v, acc_sh):
        cid = lax.axis_index("c")
        sid = lax.axis_index("s")
        r0 = pl.multiple_of(sid * rows_per_tile, 8)
        pltpu.sync_copy(zeros_hbm.at[pl.ds(r0, rows_per_tile)],
                        acc_sh.at[pl.ds(r0, rows_per_tile)])
        plsc.subcore_barrier()
        base = (cid * _NS + sid) * (chunks_per_tile * _CHUNK)

        @pl.loop(0, chunks_per_tile)
        def _(k):
            off = pl.multiple_of(base + k * _CHUNK, 8)
            pltpu.sync_copy(src_hbm.at[pl.ds(off, _CHUNK)], src_v)
            pltpu.sync_copy(dst_hbm.at[pl.ds(off, _CHUNK)], dst_v)
            pltpu.sync_copy(vals_hbm.at[src_v], rows_v)
            pltpu.sync_copy(rows_v, acc_sh.at[dst_v], add=True)

        plsc.subcore_barrier()
        pltpu.sync_copy(acc_sh.at[pl.ds(r0, rows_per_tile)],
                        out_hbm.at[cid, pl.ds(r0, rows_per_tile)])

    return seg_sum


def _dinv_from(d0, d1):
    deg = d0 + d1 + 1.0
    return lax.rsqrt(jnp.maximum(deg, 1e-12))


def _k1_body(x_ref, w_ref, d0_ref, d1_ref, g_ref):
    dinv = _dinv_from(d0_ref[...], d1_ref[...])
    g_ref[...] = jnp.dot(x_ref[...], w_ref[...],
                         preferred_element_type=jnp.float32) * dinv


def _bn_relu(t):
    mu = jnp.mean(t, axis=0, keepdims=True)
    var = jnp.mean((t - mu) ** 2, axis=0, keepdims=True)
    return jnp.maximum((t - mu) * lax.rsqrt(var + 1e-5), 0.0)


def _k2_body(s0_ref, s1_ref, g_ref, d0_ref, d1_ref, b_ref, w_ref, out_ref):
    dinv = _dinv_from(d0_ref[...], d1_ref[...])
    t = dinv * (s0_ref[...] + s1_ref[...] + g_ref[...]) + b_ref[...]
    y = _bn_relu(t)
    out_ref[...] = jnp.dot(y, w_ref[...],
                           preferred_element_type=jnp.float32) * dinv


def _k3_body(s0_ref, s1_ref, g_ref, d0_ref, d1_ref, b_ref, out_ref):
    dinv = _dinv_from(d0_ref[...], d1_ref[...])
    t = dinv * (s0_ref[...] + s1_ref[...] + g_ref[...]) + b_ref[...]
    y = _bn_relu(t)
    out_ref[...] = jnp.concatenate(
        [y, jnp.zeros((N_ACC - N, H), jnp.float32)], axis=0)


def _k4_body(pe0_ref, pe1_ref, w1_ref, b1_ref, w2_ref, b2_ref, out_ref):
    pe_a = pe0_ref[...] + pe1_ref[...]
    xc = lax.slice(pe_a, (P, 0), (P + 1, H))
    pemb = lax.slice(pe_a, (0, 0), (P, H))
    w_top = lax.slice(w1_ref[...], (0, 0), (H, H))
    w_bot = lax.slice(w1_ref[...], (H, 0), (2 * H, H))
    z = jnp.maximum(
        jnp.dot(pemb, w_bot, preferred_element_type=jnp.float32)
        + jnp.dot(xc, w_top, preferred_element_type=jnp.float32)
        + b1_ref[...], 0.0)
    out_ref[...] = jnp.dot(z, w2_ref[...],
                           preferred_element_type=jnp.float32) + b2_ref[...]


def kernel(x, edge_index, curr_node_id, partitions, core_values,
           W0, b0, W1, b1, W2, b2, lin1_W, lin1_b, lin2_W, lin2_b):
    del core_values

    f32 = jnp.float32
    i32 = jnp.int32

    src = jnp.concatenate(
        [edge_index[0], jnp.zeros((E_PAD - E,), i32)])
    dst = jnp.concatenate(
        [edge_index[1], jnp.full((E_PAD - E,), N, i32)])

    parts_pad = jnp.pad(partitions, ((0, 0), (0, S_PAD - S)),
                        constant_values=N)
    curr_row = jnp.full((1, S_PAD), N, i32).at[0, 0].set(
        jnp.asarray(curr_node_id, i32))
    pidx = jnp.concatenate(
        [jnp.concatenate([parts_pad, curr_row], axis=0).reshape(-1),
         jnp.full((PE_PAD - NPART * S_PAD,), N, i32)])
    pdst = jnp.concatenate(
        [jnp.repeat(jnp.arange(NPART, dtype=i32), S_PAD),
         jnp.full((PE_PAD - NPART * S_PAD,), PART_ACC - 1, i32)])

    zeros_node = jnp.zeros((N_ACC, H), f32)
    zeros_part = jnp.zeros((PART_ACC, H), f32)

    b0r = b0.reshape(1, H)
    b1r = b1.reshape(1, H)
    b2r = b2.reshape(1, H)
    lin1_br = lin1_b.reshape(1, H)
    lin2_br = lin2_b.reshape(1, 1)

    deg_k = _seg_sum_gather(H, N_ACC, E_PAD)
    edge_k = _seg_sum_gather(H, N_ACC, E_PAD)
    part_k = _seg_sum_gather(H, PART_ACC, PE_PAD)

    ones_rows = jnp.ones((8, H), f32)
    src_zero = jnp.zeros((E_PAD,), i32)
    degp = deg_k(ones_rows, src_zero, dst, zeros_node)
    d0 = degp[0, :N, :1]
    d1 = degp[1, :N, :1]

    g = pl.pallas_call(
        _k1_body,
        out_shape=jax.ShapeDtypeStruct((N, H), f32),
    )(x, W0, d0, d1)

    for (b_l, W_next) in ((b0r, W1), (b1r, W2)):
        s = edge_k(g, src, dst, zeros_node)
        g = pl.pallas_call(
            _k2_body,
            out_shape=jax.ShapeDtypeStruct((N, H), f32),
        )(s[0, :N], s[1, :N], g, d0, d1, b_l, W_next)

    s = edge_k(g, src, dst, zeros_node)
    h_pad = pl.pallas_call(
        _k3_body,
        out_shape=jax.ShapeDtypeStruct((N_ACC, H), f32),
    )(s[0, :N], s[1, :N], g, d0, d1, b2r)

    pe = part_k(h_pad, pidx, pdst, zeros_part)

    out = pl.pallas_call(
        _k4_body,
        out_shape=jax.ShapeDtypeStruct((P, 1), f32),
    )(pe[0], pe[1], lin1_W, lin1_br, lin2_W, lin2_br)
    return out


# speedup vs baseline: 9.3337x; 9.3337x over previous
import functools

import jax
import jax.numpy as jnp
from jax import lax
from jax.experimental import pallas as pl
from jax.experimental.pallas import tpu as pltpu
from jax.experimental.pallas import tpu_sc as plsc

N = 10000
E = 320000
H = 128
P = 64
S = 156

_NC = 2
_NS = 16
_NW = _NC * _NS
_CHUNK = 128

N_ACC = 10112
E_PAD = 323584
NPART = P + 1
S_PAD = 160
PART_ACC = 128
PE_PAD = 12288


def _seg_sum_gather(d, n_acc, e_pad):
    chunks_per_tile = e_pad // (_NW * _CHUNK)
    rows_per_tile = n_acc // _NS
    mesh = plsc.VectorSubcoreMesh(core_axis_name="c", subcore_axis_name="s")

    @functools.partial(
        pl.kernel,
        out_type=jax.ShapeDtypeStruct((_NC, n_acc, d), jnp.float32),
        mesh=mesh,
        scratch_types=[
            pltpu.VMEM((_CHUNK,), jnp.int32),
            pltpu.VMEM((_CHUNK,), jnp.int32),
            pltpu.VMEM((_CHUNK, d), jnp.float32),
            pltpu.VMEM_SHARED((n_acc, d), jnp.float32),
        ],
    )
    def seg_sum(vals_hbm, src_hbm, dst_hbm, zeros_hbm, out_hbm,
                src_v, dst_v, rows_v, acc_sh):
        cid = lax.axis_index("c")
        sid = lax.axis_index("s")
        r0 = pl.multiple_of(sid * rows_per_tile, 8)
        pltpu.sync_copy(zeros_hbm.at[pl.ds(r0, rows_per_tile)],
                        acc_sh.at[pl.ds(r0, rows_per_tile)])
        plsc.subcore_barrier()
        base = (cid * _NS + sid) * (chunks_per_tile * _CHUNK)

        @pl.loop(0, chunks_per_tile)
        def _(k):
            off = pl.multiple_of(base + k * _CHUNK, 8)
            pltpu.sync_copy(src_hbm.at[pl.ds(off, _CHUNK)], src_v)
            pltpu.sync_copy(dst_hbm.at[pl.ds(off, _CHUNK)], dst_v)
            pltpu.sync_copy(vals_hbm.at[src_v], rows_v)
            pltpu.sync_copy(rows_v, acc_sh.at[dst_v], add=True)

        plsc.subcore_barrier()
        pltpu.sync_copy(acc_sh.at[pl.ds(r0, rows_per_tile)],
                        out_hbm.at[cid, pl.ds(r0, rows_per_tile)])

    return seg_sum


def _ones_hist(d, n_acc, e_pad):
    chunks_per_tile = e_pad // (_NW * _CHUNK)
    rows_per_tile = n_acc // _NS
    mesh = plsc.VectorSubcoreMesh(core_axis_name="c", subcore_axis_name="s")

    @functools.partial(
        pl.kernel,
        out_type=jax.ShapeDtypeStruct((_NC, n_acc, d), jnp.float32),
        mesh=mesh,
        scratch_types=[
            pltpu.VMEM((_CHUNK,), jnp.int32),
            pltpu.VMEM((_CHUNK, d), jnp.float32),
            pltpu.VMEM_SHARED((n_acc, d), jnp.float32),
        ],
    )
    def ones_hist(ones_hbm, dst_hbm, zeros_hbm, out_hbm, dst_v, rows_v, acc_sh):
        cid = lax.axis_index("c")
        sid = lax.axis_index("s")
        r0 = pl.multiple_of(sid * rows_per_tile, 8)
        pltpu.sync_copy(zeros_hbm.at[pl.ds(r0, rows_per_tile)],
                        acc_sh.at[pl.ds(r0, rows_per_tile)])
        pltpu.sync_copy(ones_hbm, rows_v)
        plsc.subcore_barrier()
        base = (cid * _NS + sid) * (chunks_per_tile * _CHUNK)

        @pl.loop(0, chunks_per_tile)
        def _(k):
            off = pl.multiple_of(base + k * _CHUNK, 8)
            pltpu.sync_copy(dst_hbm.at[pl.ds(off, _CHUNK)], dst_v)
            pltpu.sync_copy(rows_v, acc_sh.at[dst_v], add=True)

        plsc.subcore_barrier()
        pltpu.sync_copy(acc_sh.at[pl.ds(r0, rows_per_tile)],
                        out_hbm.at[cid, pl.ds(r0, rows_per_tile)])

    return ones_hist


def _dinv_from(d0, d1):
    deg = d0 + d1 + 1.0
    return lax.rsqrt(jnp.maximum(deg, 1e-12))


def _k1_body(x_ref, w_ref, d0_ref, d1_ref, g_ref):
    dinv = _dinv_from(d0_ref[...], d1_ref[...])
    g_ref[...] = jnp.dot(x_ref[...], w_ref[...],
                         preferred_element_type=jnp.float32) * dinv


def _bn_relu(t):
    mu = jnp.mean(t, axis=0, keepdims=True)
    var = jnp.mean((t - mu) ** 2, axis=0, keepdims=True)
    return jnp.maximum((t - mu) * lax.rsqrt(var + 1e-5), 0.0)


def _k2_body(s0_ref, s1_ref, g_ref, d0_ref, d1_ref, b_ref, w_ref, out_ref):
    dinv = _dinv_from(d0_ref[...], d1_ref[...])
    t = dinv * (s0_ref[...] + s1_ref[...] + g_ref[...]) + b_ref[...]
    y = _bn_relu(t)
    out_ref[...] = jnp.dot(y, w_ref[...],
                           preferred_element_type=jnp.float32) * dinv


def _k3_body(s0_ref, s1_ref, g_ref, d0_ref, d1_ref, b_ref, out_ref):
    dinv = _dinv_from(d0_ref[...], d1_ref[...])
    t = dinv * (s0_ref[...] + s1_ref[...] + g_ref[...]) + b_ref[...]
    y = _bn_relu(t)
    out_ref[...] = jnp.concatenate(
        [y, jnp.zeros((N_ACC - N, H), jnp.float32)], axis=0)


def _k4_body(pe0_ref, pe1_ref, w1_ref, b1_ref, w2_ref, b2_ref, out_ref):
    pe_a = pe0_ref[...] + pe1_ref[...]
    xc = lax.slice(pe_a, (P, 0), (P + 1, H))
    pemb = lax.slice(pe_a, (0, 0), (P, H))
    w_top = lax.slice(w1_ref[...], (0, 0), (H, H))
    w_bot = lax.slice(w1_ref[...], (H, 0), (2 * H, H))
    z = jnp.maximum(
        jnp.dot(pemb, w_bot, preferred_element_type=jnp.float32)
        + jnp.dot(xc, w_top, preferred_element_type=jnp.float32)
        + b1_ref[...], 0.0)
    out_ref[...] = jnp.dot(z, w2_ref[...],
                           preferred_element_type=jnp.float32) + b2_ref[...]


def kernel(x, edge_index, curr_node_id, partitions, core_values,
           W0, b0, W1, b1, W2, b2, lin1_W, lin1_b, lin2_W, lin2_b):
    del core_values

    f32 = jnp.float32
    i32 = jnp.int32

    src = jnp.concatenate(
        [edge_index[0], jnp.zeros((E_PAD - E,), i32)])
    dst = jnp.concatenate(
        [edge_index[1], jnp.full((E_PAD - E,), N, i32)])

    parts_pad = jnp.pad(partitions, ((0, 0), (0, S_PAD - S)),
                        constant_values=N)
    curr_row = jnp.full((1, S_PAD), N, i32).at[0, 0].set(
        jnp.asarray(curr_node_id, i32))
    pidx = jnp.concatenate(
        [jnp.concatenate([parts_pad, curr_row], axis=0).reshape(-1),
         jnp.full((PE_PAD - NPART * S_PAD,), N, i32)])
    pdst = jnp.concatenate(
        [jnp.repeat(jnp.arange(NPART, dtype=i32), S_PAD),
         jnp.full((PE_PAD - NPART * S_PAD,), PART_ACC - 1, i32)])

    zeros_node = jnp.zeros((N_ACC, H), f32)
    zeros_part = jnp.zeros((PART_ACC, H), f32)

    b0r = b0.reshape(1, H)
    b1r = b1.reshape(1, H)
    b2r = b2.reshape(1, H)
    lin1_br = lin1_b.reshape(1, H)
    lin2_br = lin2_b.reshape(1, 1)

    deg_k = _ones_hist(H, N_ACC, E_PAD)
    edge_k = _seg_sum_gather(H, N_ACC, E_PAD)
    part_k = _seg_sum_gather(H, PART_ACC, PE_PAD)

    ones_rows = jnp.ones((_CHUNK, H), f32)
    degp = deg_k(ones_rows, dst, zeros_node)
    d0 = degp[0, :N, :1]
    d1 = degp[1, :N, :1]

    g = pl.pallas_call(
        _k1_body,
        out_shape=jax.ShapeDtypeStruct((N, H), f32),
    )(x, W0, d0, d1)

    for (b_l, W_next) in ((b0r, W1), (b1r, W2)):
        s = edge_k(g, src, dst, zeros_node)
        g = pl.pallas_call(
            _k2_body,
            out_shape=jax.ShapeDtypeStruct((N, H), f32),
        )(s[0, :N], s[1, :N], g, d0, d1, b_l, W_next)

    s = edge_k(g, src, dst, zeros_node)
    h_pad = pl.pallas_call(
        _k3_body,
        out_shape=jax.ShapeDtypeStruct((N_ACC, H), f32),
    )(s[0, :N], s[1, :N], g, d0, d1, b2r)

    pe = part_k(h_pad, pidx, pdst, zeros_part)

    out = pl.pallas_call(
        _k4_body,
        out_shape=jax.ShapeDtypeStruct((P, 1), f32),
    )(pe[0], pe[1], lin1_W, lin1_br, lin2_W, lin2_br)
    return out


# speedup vs baseline: 22.6028x; 2.4216x over previous
import functools

import jax
import jax.numpy as jnp
from jax import lax
from jax.experimental import pallas as pl
from jax.experimental.pallas import tpu as pltpu
from jax.experimental.pallas import tpu_sc as plsc

N = 10000
E = 320000
H = 128
P = 64
S = 156

_NC = 2
_NS = 16
_NW = _NC * _NS
_CHUNK = 128

N_ACC = 10112
E_CPT = 80
E_PAD = _NW * E_CPT * _CHUNK
NPART = P + 1
S_PAD = 160
PART_ACC = 128
PE_CPT = 8
PE_PAD = _NW * PE_CPT * _CHUNK


def _seg_sum_gather(d, n_acc, cpt):
    rows_per_tile = n_acc // _NS
    n_half = 2 if cpt >= 16 else 1
    half = cpt // n_half
    mesh = plsc.VectorSubcoreMesh(core_axis_name="c", subcore_axis_name="s")

    @functools.partial(
        pl.kernel,
        out_type=jax.ShapeDtypeStruct((_NC, n_acc, d), jnp.float32),
        mesh=mesh,
        scratch_types=[
            pltpu.VMEM((half, _CHUNK), jnp.int32),
            pltpu.VMEM((half, _CHUNK), jnp.int32),
            pltpu.VMEM((_CHUNK, d), jnp.float32),
            pltpu.VMEM((_CHUNK, d), jnp.float32),
            pltpu.VMEM_SHARED((n_acc, d), jnp.float32),
            pltpu.SemaphoreType.DMA,
            pltpu.SemaphoreType.DMA,
        ],
    )
    def seg_sum(vals_hbm, src_hbm, dst_hbm, zeros_hbm, out_hbm,
                src_v, dst_v, rows_a, rows_b, acc_sh, sem_a, sem_b):
        cid = lax.axis_index("c")
        sid = lax.axis_index("s")
        r0 = pl.multiple_of(sid * rows_per_tile, 8)
        c0 = pl.multiple_of((cid * _NS + sid) * cpt, 8)
        pltpu.sync_copy(zeros_hbm.at[pl.ds(r0, rows_per_tile)],
                        acc_sh.at[pl.ds(r0, rows_per_tile)])
        plsc.subcore_barrier()

        for h in range(n_half):
            c0h = pl.multiple_of(c0 + h * half, 8)
            pltpu.sync_copy(src_hbm.at[pl.ds(c0h, half)], src_v)
            pltpu.sync_copy(dst_hbm.at[pl.ds(c0h, half)], dst_v)

            pltpu.async_copy(vals_hbm.at[src_v.at[0]], rows_a, sem_a)

            @pl.loop(0, half // 2 - 1)
            def _(p):
                k = p * 2
                pltpu.make_async_copy(vals_hbm.at[src_v.at[k]], rows_a,
                                      sem_a).wait()
                pltpu.async_copy(vals_hbm.at[src_v.at[k + 1]], rows_b, sem_b)
                pltpu.sync_copy(rows_a, acc_sh.at[dst_v.at[k]], add=True)
                pltpu.make_async_copy(vals_hbm.at[src_v.at[k + 1]], rows_b,
                                      sem_b).wait()
                pltpu.async_copy(vals_hbm.at[src_v.at[k + 2]], rows_a, sem_a)
                pltpu.sync_copy(rows_b, acc_sh.at[dst_v.at[k + 1]], add=True)

            k_last = half - 2
            pltpu.make_async_copy(vals_hbm.at[src_v.at[k_last]], rows_a,
                                  sem_a).wait()
            pltpu.async_copy(vals_hbm.at[src_v.at[k_last + 1]], rows_b, sem_b)
            pltpu.sync_copy(rows_a, acc_sh.at[dst_v.at[k_last]], add=True)
            pltpu.make_async_copy(vals_hbm.at[src_v.at[k_last + 1]], rows_b,
                                  sem_b).wait()
            pltpu.sync_copy(rows_b, acc_sh.at[dst_v.at[k_last + 1]], add=True)

        plsc.subcore_barrier()
        pltpu.sync_copy(acc_sh.at[pl.ds(r0, rows_per_tile)],
                        out_hbm.at[cid, pl.ds(r0, rows_per_tile)])

    return seg_sum


def _ones_hist(d, n_acc, cpt):
    rows_per_tile = n_acc // _NS
    mesh = plsc.VectorSubcoreMesh(core_axis_name="c", subcore_axis_name="s")

    @functools.partial(
        pl.kernel,
        out_type=jax.ShapeDtypeStruct((_NC, n_acc, d), jnp.float32),
        mesh=mesh,
        scratch_types=[
            pltpu.VMEM((cpt, _CHUNK), jnp.int32),
            pltpu.VMEM((_CHUNK, d), jnp.float32),
            pltpu.VMEM_SHARED((n_acc, d), jnp.float32),
        ],
    )
    def ones_hist(ones_hbm, dst_hbm, zeros_hbm, out_hbm, dst_v, rows_v, acc_sh):
        cid = lax.axis_index("c")
        sid = lax.axis_index("s")
        r0 = pl.multiple_of(sid * rows_per_tile, 8)
        c0 = pl.multiple_of((cid * _NS + sid) * cpt, 8)
        pltpu.sync_copy(dst_hbm.at[pl.ds(c0, cpt)], dst_v)
        pltpu.sync_copy(zeros_hbm.at[pl.ds(r0, rows_per_tile)],
                        acc_sh.at[pl.ds(r0, rows_per_tile)])
        pltpu.sync_copy(ones_hbm, rows_v)
        plsc.subcore_barrier()

        @pl.loop(0, cpt)
        def _(k):
            pltpu.sync_copy(rows_v, acc_sh.at[dst_v.at[k]], add=True)

        plsc.subcore_barrier()
        pltpu.sync_copy(acc_sh.at[pl.ds(r0, rows_per_tile)],
                        out_hbm.at[cid, pl.ds(r0, rows_per_tile)])

    return ones_hist


def _dinv_from(d0, d1):
    deg = d0 + d1 + 1.0
    return lax.rsqrt(jnp.maximum(deg, 1e-12))


def _k1_body(x_ref, w_ref, d0_ref, d1_ref, g_ref):
    dinv = _dinv_from(d0_ref[...], d1_ref[...])
    g_ref[...] = jnp.dot(x_ref[...], w_ref[...],
                         preferred_element_type=jnp.float32) * dinv


def _bn_relu(t):
    mu = jnp.mean(t, axis=0, keepdims=True)
    var = jnp.mean((t - mu) ** 2, axis=0, keepdims=True)
    return jnp.maximum((t - mu) * lax.rsqrt(var + 1e-5), 0.0)


def _k2_body(s0_ref, s1_ref, g_ref, d0_ref, d1_ref, b_ref, w_ref, out_ref):
    dinv = _dinv_from(d0_ref[...], d1_ref[...])
    t = dinv * (s0_ref[...] + s1_ref[...] + g_ref[...]) + b_ref[...]
    y = _bn_relu(t)
    out_ref[...] = jnp.dot(y, w_ref[...],
                           preferred_element_type=jnp.float32) * dinv


def _k3_body(s0_ref, s1_ref, g_ref, d0_ref, d1_ref, b_ref, out_ref):
    dinv = _dinv_from(d0_ref[...], d1_ref[...])
    t = dinv * (s0_ref[...] + s1_ref[...] + g_ref[...]) + b_ref[...]
    y = _bn_relu(t)
    out_ref[...] = jnp.concatenate(
        [y, jnp.zeros((N_ACC - N, H), jnp.float32)], axis=0)


def _k4_body(pe0_ref, pe1_ref, w1_ref, b1_ref, w2_ref, b2_ref, out_ref):
    pe_a = pe0_ref[...] + pe1_ref[...]
    xc = lax.slice(pe_a, (P, 0), (P + 1, H))
    pemb = lax.slice(pe_a, (0, 0), (P, H))
    w_top = lax.slice(w1_ref[...], (0, 0), (H, H))
    w_bot = lax.slice(w1_ref[...], (H, 0), (2 * H, H))
    z = jnp.maximum(
        jnp.dot(pemb, w_bot, preferred_element_type=jnp.float32)
        + jnp.dot(xc, w_top, preferred_element_type=jnp.float32)
        + b1_ref[...], 0.0)
    out_ref[...] = jnp.dot(z, w2_ref[...],
                           preferred_element_type=jnp.float32) + b2_ref[...]


def kernel(x, edge_index, curr_node_id, partitions, core_values,
           W0, b0, W1, b1, W2, b2, lin1_W, lin1_b, lin2_W, lin2_b):
    del core_values

    f32 = jnp.float32
    i32 = jnp.int32

    ep = E_PAD - E
    src = jnp.concatenate(
        [edge_index[0], jnp.arange(ep, dtype=i32) % N]).reshape(-1, _CHUNK)
    dst = jnp.concatenate(
        [edge_index[1],
         N + jnp.arange(ep, dtype=i32) % (N_ACC - N)]).reshape(-1, _CHUNK)

    zpad = N + jnp.arange(NPART * (S_PAD - S), dtype=i32) % (N_ACC - N)
    parts_pad = jnp.concatenate(
        [jnp.concatenate([partitions,
                          (N + jnp.arange(S, dtype=i32) % (N_ACC - N))
                          .at[0].set(jnp.asarray(curr_node_id, i32))
                          .reshape(1, S)], axis=0),
         zpad.reshape(NPART, S_PAD - S)], axis=1)
    ptail = PE_PAD - NPART * S_PAD
    pidx = jnp.concatenate(
        [parts_pad.reshape(-1),
         jnp.arange(ptail, dtype=i32) % N]).reshape(-1, _CHUNK)
    pdst = jnp.concatenate(
        [jnp.repeat(jnp.arange(NPART, dtype=i32), S_PAD),
         NPART + jnp.arange(ptail, dtype=i32) % (PART_ACC - NPART)]
    ).reshape(-1, _CHUNK)

    zeros_node = jnp.zeros((N_ACC, H), f32)
    zeros_part = jnp.zeros((PART_ACC, H), f32)

    b0r = b0.reshape(1, H)
    b1r = b1.reshape(1, H)
    b2r = b2.reshape(1, H)
    lin1_br = lin1_b.reshape(1, H)
    lin2_br = lin2_b.reshape(1, 1)

    deg_k = _ones_hist(H, N_ACC, E_CPT)
    edge_k = _seg_sum_gather(H, N_ACC, E_CPT)
    part_k = _seg_sum_gather(H, PART_ACC, PE_CPT)

    ones_rows = jnp.ones((_CHUNK, H), f32)
    degp = deg_k(ones_rows, dst, zeros_node)
    d0 = degp[0, :N, :1]
    d1 = degp[1, :N, :1]

    g = pl.pallas_call(
        _k1_body,
        out_shape=jax.ShapeDtypeStruct((N, H), f32),
    )(x, W0, d0, d1)

    for (b_l, W_next) in ((b0r, W1), (b1r, W2)):
        s = edge_k(g, src, dst, zeros_node)
        g = pl.pallas_call(
            _k2_body,
            out_shape=jax.ShapeDtypeStruct((N, H), f32),
        )(s[0, :N], s[1, :N], g, d0, d1, b_l, W_next)

    s = edge_k(g, src, dst, zeros_node)
    h_pad = pl.pallas_call(
        _k3_body,
        out_shape=jax.ShapeDtypeStruct((N_ACC, H), f32),
    )(s[0, :N], s[1, :N], g, d0, d1, b2r)

    pe = part_k(h_pad, pidx, pdst, zeros_part)

    out = pl.pallas_call(
        _k4_body,
        out_shape=jax.ShapeDtypeStruct((P, 1), f32),
    )(pe[0], pe[1], lin1_W, lin1_br, lin2_W, lin2_br)
    return out
